# baseline (speedup 1.0000x reference)
import functools

import jax
import jax.numpy as jnp
from jax import lax
from jax.experimental import pallas as pl
from jax.experimental.pallas import tpu as pltpu
from jax.experimental.pallas import tpu_sc as plsc


def _segsum_sc(flat, BS, P, D):
    info = plsc.get_sparse_core_info()
    NC, NS, L = info.num_cores, info.num_subcores, info.num_lanes
    NW = NC * NS
    NV = D // L
    BPW = BS // NW
    RPC = 256
    SPC = RPC // P
    NCHUNK = (BPW * P) // RPC
    NBUF = 3
    mesh = plsc.VectorSubcoreMesh(core_axis_name="c", subcore_axis_name="s")

    @functools.partial(
        pl.kernel,
        out_type=jax.ShapeDtypeStruct((BS, D), jnp.float32),
        mesh=mesh,
        cost_estimate=pl.CostEstimate(
            flops=BS * P * D,
            transcendentals=0,
            bytes_accessed=(BS * P * D + BS * D) * 4,
        ),
        scratch_types=[
            [pltpu.VMEM((RPC, D), jnp.float32)] * NBUF,
            pltpu.VMEM((BPW, D), jnp.float32),
            [pltpu.SemaphoreType.DMA] * NBUF,
        ],
    )
    def seg(flat_hbm, out_hbm, bufs, res, hsems):
        sid = lax.axis_index("s")
        wid = lax.axis_index("c") * NS + sid
        in_base = wid * (BPW * P)

        def hbm_start(g):
            return pltpu.async_copy(
                flat_hbm.at[pl.ds(in_base + g * RPC, RPC)],
                bufs[g % NBUF], hsems[g % NBUF])

        def reduce_chunk(buf, g):
            def seg_body(t, _):
                base = t * P
                acc = [buf[base, pl.ds(j * L, L)] for j in range(NV)]
                def row_body(r, acc):
                    return tuple(
                        acc[j] + buf[base + r, pl.ds(j * L, L)]
                        for j in range(NV)
                    )
                acc = lax.fori_loop(1, P, row_body, tuple(acc))
                for j in range(NV):
                    res[g * SPC + t, pl.ds(j * L, L)] = acc[j]
                return _
            lax.fori_loop(0, SPC, seg_body, 0)

        hbm_d = [hbm_start(0), hbm_start(1)]
        for gg in range(0, NCHUNK, NBUF):
            for b in range(NBUF):
                g = gg + b
                if g >= NCHUNK:
                    break
                hbm_d.pop(0).wait()
                if g + 2 < NCHUNK:
                    hbm_d.append(hbm_start(g + 2))
                reduce_chunk(bufs[g % NBUF], g)

        pltpu.sync_copy(res, out_hbm.at[pl.ds(wid * BPW, BPW)])

    return seg(flat)


def _linear(s, w, bias):
    y = jax.lax.dot_general(
        s, w, (((1,), (1,)), ((), ())),
        preferred_element_type=jnp.float32,
        precision=jax.lax.Precision.HIGHEST,
    )
    return y + bias


def _mlp_body(s_ref, w_ref, b_ref, o_ref):
    o_ref[...] = _linear(s_ref[...], w_ref[...], b_ref[...])


def _fused_body(x_ref, w_ref, b_ref, o_ref):
    s = jnp.sum(x_ref[...], axis=1)
    o_ref[...] = _linear(s, w_ref[...], b_ref[...])


def kernel(pert_batch, W, b):
    B, P, D = pert_batch.shape
    OUT = W.shape[0]
    BS = 1024
    BB = 512
    bias = (P * b).reshape(1, OUT)
    flat = pert_batch.reshape(B * P, D)

    s_sc = _segsum_sc(flat, BS, P, D)

    nblk = (B - BS) // BB
    y_tc = pl.pallas_call(
        _fused_body,
        grid=(nblk,),
        in_specs=[
            pl.BlockSpec((BB, P, D), lambda i, o=BS // BB: (o + i, 0, 0)),
            pl.BlockSpec((OUT, D), lambda i: (0, 0)),
            pl.BlockSpec((1, OUT), lambda i: (0, 0)),
        ],
        out_specs=pl.BlockSpec((BB, OUT), lambda i: (i, 0)),
        out_shape=jax.ShapeDtypeStruct((B - BS, OUT), jnp.float32),
    )(pert_batch, W, bias)

    y_sc = pl.pallas_call(
        _mlp_body,
        in_specs=[
            pl.BlockSpec((BS, D), lambda: (0, 0)),
            pl.BlockSpec((OUT, D), lambda: (0, 0)),
            pl.BlockSpec((1, OUT), lambda: (0, 0)),
        ],
        out_specs=pl.BlockSpec((BS, OUT), lambda: (0, 0)),
        out_shape=jax.ShapeDtypeStruct((BS, OUT), jnp.float32),
    )(s_sc, W, bias)

    return jnp.concatenate([y_sc, y_tc], axis=0)


# speedup vs baseline: 1.0001x; 1.0001x over previous
import functools

import jax
import jax.numpy as jnp
from jax import lax
from jax.experimental import pallas as pl
from jax.experimental.pallas import tpu as pltpu
from jax.experimental.pallas import tpu_sc as plsc


def _segsum_sc(flat, BS, P, D):
    info = plsc.get_sparse_core_info()
    NC, NS, L = info.num_cores, info.num_subcores, info.num_lanes
    NW = NC * NS
    NV = D // L
    BPW = BS // NW
    RPC = 256
    SPC = RPC // P
    NCHUNK = (BPW * P) // RPC
    NBUF = 3
    mesh = plsc.VectorSubcoreMesh(core_axis_name="c", subcore_axis_name="s")

    @functools.partial(
        pl.kernel,
        out_type=jax.ShapeDtypeStruct((BS, D), jnp.float32),
        mesh=mesh,
        cost_estimate=pl.CostEstimate(
            flops=BS * P * D,
            transcendentals=0,
            bytes_accessed=(BS * P * D + BS * D) * 4,
        ),
        scratch_types=[
            [pltpu.VMEM((RPC, D), jnp.float32)] * NBUF,
            pltpu.VMEM((BPW, D), jnp.float32),
            [pltpu.SemaphoreType.DMA] * NBUF,
        ],
    )
    def seg(flat_hbm, out_hbm, bufs, res, hsems):
        sid = lax.axis_index("s")
        wid = lax.axis_index("c") * NS + sid
        in_base = wid * (BPW * P)

        def hbm_start(g):
            return pltpu.async_copy(
                flat_hbm.at[pl.ds(in_base + g * RPC, RPC)],
                bufs[g % NBUF], hsems[g % NBUF])

        def reduce_chunk(buf, g):
            def seg_body(t, _):
                base = t * P
                acc = [buf[base, pl.ds(j * L, L)] for j in range(NV)]
                def row_body(r, acc):
                    return tuple(
                        acc[j] + buf[base + r, pl.ds(j * L, L)]
                        for j in range(NV)
                    )
                acc = lax.fori_loop(1, P, row_body, tuple(acc))
                for j in range(NV):
                    res[g * SPC + t, pl.ds(j * L, L)] = acc[j]
                return _
            lax.fori_loop(0, SPC, seg_body, 0)

        hbm_d = [hbm_start(0), hbm_start(1)]
        for gg in range(0, NCHUNK, NBUF):
            for b in range(NBUF):
                g = gg + b
                if g >= NCHUNK:
                    break
                hbm_d.pop(0).wait()
                if g + 2 < NCHUNK:
                    hbm_d.append(hbm_start(g + 2))
                reduce_chunk(bufs[g % NBUF], g)

        pltpu.sync_copy(res, out_hbm.at[pl.ds(wid * BPW, BPW)])

    return seg(flat)


def _linear(s, w, bias):
    y = jax.lax.dot_general(
        s, w, (((1,), (1,)), ((), ())),
        preferred_element_type=jnp.float32,
        precision=jax.lax.Precision.HIGHEST,
    )
    return y + bias


def _mlp_body(s_ref, w_ref, b_ref, o_ref):
    o_ref[...] = _linear(s_ref[...], w_ref[...], b_ref[...])


def _fused_body(x_ref, w_ref, b_ref, o_ref):
    s = jnp.sum(x_ref[...], axis=1)
    o_ref[...] = _linear(s, w_ref[...], b_ref[...])


def kernel(pert_batch, W, b):
    B, P, D = pert_batch.shape
    OUT = W.shape[0]
    BS = 1024
    BB = 512
    bias = (P * b).reshape(1, OUT)
    flat = pert_batch.reshape(B * P, D)

    s_sc = _segsum_sc(flat, BS, P, D)

    nblk = (B - BS) // BB
    y_tc = pl.pallas_call(
        _fused_body,
        grid=(nblk,),
        in_specs=[
            pl.BlockSpec((BB, P, D), lambda i, o=BS // BB: (o + i, 0, 0)),
            pl.BlockSpec((OUT, D), lambda i: (0, 0)),
            pl.BlockSpec((1, OUT), lambda i: (0, 0)),
        ],
        out_specs=pl.BlockSpec((BB, OUT), lambda i: (i, 0)),
        out_shape=jax.ShapeDtypeStruct((B - BS, OUT), jnp.float32),
        cost_estimate=pl.CostEstimate(
            flops=2 * (B - BS) * (P + OUT) * D,
            transcendentals=0,
            bytes_accessed=(B - BS) * (P + 1) * D * 4,
        ),
    )(pert_batch, W, bias)

    s_sc, y_tc = jax.lax.optimization_barrier((s_sc, y_tc))

    y_sc = pl.pallas_call(
        _mlp_body,
        in_specs=[
            pl.BlockSpec((BS, D), lambda: (0, 0)),
            pl.BlockSpec((OUT, D), lambda: (0, 0)),
            pl.BlockSpec((1, OUT), lambda: (0, 0)),
        ],
        out_specs=pl.BlockSpec((BS, OUT), lambda: (0, 0)),
        out_shape=jax.ShapeDtypeStruct((BS, OUT), jnp.float32),
    )(s_sc, W, bias)

    return jnp.concatenate([y_sc, y_tc], axis=0)
